# baseline (speedup 1.0000x reference)
import functools

import jax
import jax.numpy as jnp
from jax import lax
from jax.experimental import pallas as pl
from jax.experimental.pallas import tpu as pltpu
from jax.experimental.pallas import tpu_sc as plsc

_B = 16384
_D = 64
_L = 16
_NC = 2
_NS = 16
_NW = _NC * _NS
_BPW = _B // _NW
_CHUNK = 64
_NCHUNK = _BPW // _CHUNK
_NBLK = _D // _L


def _sc_gather_score(users, pos, neg, t0, t1, t2, item):
    mesh = plsc.VectorSubcoreMesh(core_axis_name="c", subcore_axis_name="s")

    @functools.partial(
        pl.kernel,
        out_type=(
            jax.ShapeDtypeStruct((_B, _L), jnp.float32),
            jax.ShapeDtypeStruct((_NW, _L), jnp.float32),
        ),
        mesh=mesh,
        compiler_params=pltpu.CompilerParams(use_tc_tiling_on_sc=True),
        scratch_types=[
            pltpu.VMEM((_BPW,), jnp.int32),
            pltpu.VMEM((_BPW,), jnp.int32),
            pltpu.VMEM((_BPW,), jnp.int32),
            pltpu.VMEM((_CHUNK, _D), jnp.float32),
            pltpu.VMEM((_CHUNK, _D), jnp.float32),
            pltpu.VMEM((_CHUNK, _D), jnp.float32),
            pltpu.VMEM((_CHUNK, _D), jnp.float32),
            pltpu.VMEM((_CHUNK, _D), jnp.float32),
            pltpu.VMEM((_BPW, _L), jnp.float32),
            pltpu.VMEM((_L,), jnp.float32),
            pltpu.SemaphoreType.DMA,
        ],
    )
    def sc_body(users_h, pos_h, neg_h, t0_h, t1_h, t2_h, item_h,
                q_h, r_h,
                uidx, pidx, nidx, t0v, t1v, t2v, pv, nv, qv, regv, sem):
        wid = lax.axis_index("s") * _NC + lax.axis_index("c")
        base = wid * _BPW
        pltpu.sync_copy(users_h.at[pl.ds(base, _BPW)], uidx)
        pltpu.sync_copy(pos_h.at[pl.ds(base, _BPW)], pidx)
        pltpu.sync_copy(neg_h.at[pl.ds(base, _BPW)], nidx)

        def chunk_body(ci, regacc):
            off = ci * _CHUNK
            for g in range(_CHUNK // _L):
                uvec = uidx[pl.ds(off + g * _L, _L)]
                pvec = pidx[pl.ds(off + g * _L, _L)]
                nvec = nidx[pl.ds(off + g * _L, _L)]
                for l in range(_L):
                    j = g * _L + l
                    pltpu.async_copy(t0_h.at[uvec[l]], t0v.at[j], sem)
                    pltpu.async_copy(t1_h.at[uvec[l]], t1v.at[j], sem)
                    pltpu.async_copy(t2_h.at[uvec[l]], t2v.at[j], sem)
                    pltpu.async_copy(item_h.at[pvec[l]], pv.at[j], sem)
                    pltpu.async_copy(item_h.at[nvec[l]], nv.at[j], sem)
            pltpu.make_async_copy(t0_h.at[pl.ds(0, _CHUNK)], t0v, sem).wait()
            pltpu.make_async_copy(t1_h.at[pl.ds(0, _CHUNK)], t1v, sem).wait()
            pltpu.make_async_copy(t2_h.at[pl.ds(0, _CHUNK)], t2v, sem).wait()
            pltpu.make_async_copy(item_h.at[pl.ds(0, _CHUNK)], pv, sem).wait()
            pltpu.make_async_copy(item_h.at[pl.ds(0, _CHUNK)], nv, sem).wait()

            def row_body(r, reg):
                q = None
                for k in range(_NBLK):
                    sl = pl.ds(k * _L, _L)
                    a0 = t0v[r, sl]
                    a1 = t1v[r, sl]
                    a2 = t2v[r, sl]
                    pp = pv[r, sl]
                    nn = nv[r, sl]
                    u = a0 + 0.5 * (a1 + a2)
                    term = u * (nn - pp)
                    q = term if q is None else q + term
                    reg = reg + (u * u + pp * pp + nn * nn)
                qv[off + r, :] = q
                return reg

            return lax.fori_loop(0, _CHUNK, row_body, regacc)

        regacc = lax.fori_loop(0, _NCHUNK, chunk_body,
                               jnp.zeros((_L,), jnp.float32))
        pltpu.sync_copy(qv, q_h.at[pl.ds(base, _BPW)])
        regv[...] = regacc
        pltpu.sync_copy(regv, r_h.at[wid])

    return sc_body(users, pos, neg, t0, t1, t2, item)


def _tc_finalize(q, r):
    def tc_body(q_ref, r_ref, loss_ref, reg_ref):
        d = jnp.sum(q_ref[...], axis=1)
        loss_ref[0, 0] = jnp.mean(jax.nn.softplus(d))
        reg_ref[0, 0] = 0.5 * jnp.sum(r_ref[...]) / float(_B)

    loss, reg = pl.pallas_call(
        tc_body,
        out_shape=(
            jax.ShapeDtypeStruct((1, 1), jnp.float32),
            jax.ShapeDtypeStruct((1, 1), jnp.float32),
        ),
        out_specs=(
            pl.BlockSpec(memory_space=pltpu.SMEM),
            pl.BlockSpec(memory_space=pltpu.SMEM),
        ),
    )(q, r)
    return loss[0, 0], reg[0, 0]


def kernel(users, pos, neg, user_table_0, user_table_1, user_table_2,
           item_table):
    q, r = _sc_gather_score(users, pos, neg, user_table_0, user_table_1,
                            user_table_2, item_table)
    loss, reg_loss = _tc_finalize(q, r)
    return (loss, reg_loss)


# speedup vs baseline: 1.5650x; 1.5650x over previous
import functools

import jax
import jax.numpy as jnp
from jax import lax
from jax.experimental import pallas as pl
from jax.experimental.pallas import tpu as pltpu
from jax.experimental.pallas import tpu_sc as plsc

_B = 16384
_V = 100000
_D = 64
_L = 16
_NC = 2
_NS = 16
_NW = _NC * _NS
_BPW = _B // _NW
_CHUNK = 64
_NCHUNK = _BPW // _CHUNK
_NBLK = _D // _L
_CB = 2048


def _tc_pack(t0, t1, t2, item):
    t0t, t1t, t2t, itt = t0.T, t1.T, t2.T, item.T

    def pack_body(a0, a1, a2, it, u_ref, i_ref):
        u = a0[...] + 0.5 * (a1[...] + a2[...])
        u_ref[...] = u.T
        i_ref[...] = it[...].T

    grid = (pl.cdiv(_V, _CB),)
    in_spec = pl.BlockSpec((_D, _CB), lambda j: (0, j))
    out_spec = pl.BlockSpec((_CB, _D), lambda j: (j, 0))
    return pl.pallas_call(
        pack_body,
        grid=grid,
        in_specs=[in_spec] * 4,
        out_specs=[out_spec] * 2,
        out_shape=[jax.ShapeDtypeStruct((_V, _D), jnp.float32)] * 2,
    )(t0t, t1t, t2t, itt)


def _sc_gather_score(users, pos, neg, utab, itab):
    mesh = plsc.VectorSubcoreMesh(core_axis_name="c", subcore_axis_name="s")

    @functools.partial(
        pl.kernel,
        out_type=(
            jax.ShapeDtypeStruct((_B, _L), jnp.float32),
            jax.ShapeDtypeStruct((_NW, _L), jnp.float32),
        ),
        mesh=mesh,
        compiler_params=pltpu.CompilerParams(use_tc_tiling_on_sc=True),
        scratch_types=[
            pltpu.VMEM((_BPW,), jnp.int32),
            pltpu.VMEM((_BPW,), jnp.int32),
            pltpu.VMEM((_BPW,), jnp.int32),
            pltpu.VMEM((_CHUNK, _D), jnp.float32),
            pltpu.VMEM((_CHUNK, _D), jnp.float32),
            pltpu.VMEM((_CHUNK, _D), jnp.float32),
            pltpu.VMEM((_BPW, _L), jnp.float32),
            pltpu.VMEM((_L,), jnp.float32),
            pltpu.SemaphoreType.DMA,
        ],
    )
    def sc_body(users_h, pos_h, neg_h, utab_h, itab_h,
                q_h, r_h,
                uidx, pidx, nidx, uv, pv, nv, qv, regv, sem):
        wid = lax.axis_index("s") * _NC + lax.axis_index("c")
        base = wid * _BPW
        pltpu.sync_copy(users_h.at[pl.ds(base, _BPW)], uidx)
        pltpu.sync_copy(pos_h.at[pl.ds(base, _BPW)], pidx)
        pltpu.sync_copy(neg_h.at[pl.ds(base, _BPW)], nidx)

        def chunk_body(ci, regacc):
            off = ci * _CHUNK
            for g in range(_CHUNK // _L):
                uvec = uidx[pl.ds(off + g * _L, _L)]
                pvec = pidx[pl.ds(off + g * _L, _L)]
                nvec = nidx[pl.ds(off + g * _L, _L)]
                for l in range(_L):
                    j = g * _L + l
                    pltpu.async_copy(utab_h.at[uvec[l]], uv.at[j], sem)
                    pltpu.async_copy(itab_h.at[pvec[l]], pv.at[j], sem)
                    pltpu.async_copy(itab_h.at[nvec[l]], nv.at[j], sem)
            pltpu.make_async_copy(utab_h.at[pl.ds(0, _CHUNK)], uv, sem).wait()
            pltpu.make_async_copy(itab_h.at[pl.ds(0, _CHUNK)], pv, sem).wait()
            pltpu.make_async_copy(itab_h.at[pl.ds(0, _CHUNK)], nv, sem).wait()

            def row_body(r, reg):
                q = None
                for k in range(_NBLK):
                    sl = pl.ds(k * _L, _L)
                    u = uv[r, sl]
                    pp = pv[r, sl]
                    nn = nv[r, sl]
                    term = u * (nn - pp)
                    q = term if q is None else q + term
                    reg = reg + (u * u + pp * pp + nn * nn)
                qv[off + r, :] = q
                return reg

            return lax.fori_loop(0, _CHUNK, row_body, regacc)

        regacc = lax.fori_loop(0, _NCHUNK, chunk_body,
                               jnp.zeros((_L,), jnp.float32))
        pltpu.sync_copy(qv, q_h.at[pl.ds(base, _BPW)])
        regv[...] = regacc
        pltpu.sync_copy(regv, r_h.at[wid])

    return sc_body(users, pos, neg, utab, itab)


def _tc_finalize(q, r):
    def tc_body(q_ref, r_ref, loss_ref, reg_ref):
        d = jnp.sum(q_ref[...], axis=1)
        loss_ref[0, 0] = jnp.mean(jax.nn.softplus(d))
        reg_ref[0, 0] = 0.5 * jnp.sum(r_ref[...]) / float(_B)

    loss, reg = pl.pallas_call(
        tc_body,
        out_shape=(
            jax.ShapeDtypeStruct((1, 1), jnp.float32),
            jax.ShapeDtypeStruct((1, 1), jnp.float32),
        ),
        out_specs=(
            pl.BlockSpec(memory_space=pltpu.SMEM),
            pl.BlockSpec(memory_space=pltpu.SMEM),
        ),
    )(q, r)
    return loss[0, 0], reg[0, 0]


def kernel(users, pos, neg, user_table_0, user_table_1, user_table_2,
           item_table):
    utab, itab = _tc_pack(user_table_0, user_table_1, user_table_2,
                          item_table)
    q, r = _sc_gather_score(users, pos, neg, utab, itab)
    loss, reg_loss = _tc_finalize(q, r)
    return (loss, reg_loss)


# speedup vs baseline: 1.7367x; 1.1098x over previous
import functools

import jax
import jax.numpy as jnp
from jax import lax
from jax.experimental import pallas as pl
from jax.experimental.pallas import tpu as pltpu
from jax.experimental.pallas import tpu_sc as plsc

_B = 16384
_V = 100000
_D = 64
_L = 16
_NC = 2
_NS = 16
_NW = _NC * _NS
_BPW = _B // _NW
_CHUNK = 64
_NCHUNK = _BPW // _CHUNK
_NBLK = _D // _L
_CBH = 2048
_VH = 51200


def _tc_pack(t0, t1, t2, item):
    t0t, t1t, t2t, itt = t0.T, t1.T, t2.T, item.T

    def pack_body(a0l, a0h, a1l, a1h, a2l, a2h, itl, ith, u_ref, i_ref):
        ul = a0l[...] + 0.5 * (a1l[...] + a2l[...])
        uh = a0h[...] + 0.5 * (a1h[...] + a2h[...])
        u_ref[...] = jnp.concatenate([ul.T, uh.T], axis=1)
        i_ref[...] = jnp.concatenate([itl[...].T, ith[...].T], axis=1)

    nblk = _VH // _CBH
    last_in_blk = pl.cdiv(_V, _CBH) - 1
    lo_spec = pl.BlockSpec((_D, _CBH), lambda j: (0, j))
    hi_spec = pl.BlockSpec(
        (_D, _CBH), lambda j: (0, jnp.minimum(j + nblk, last_in_blk)))
    out_spec = pl.BlockSpec((_CBH, 2 * _D), lambda j: (j, 0))
    return pl.pallas_call(
        pack_body,
        grid=(nblk,),
        in_specs=[lo_spec, hi_spec] * 4,
        out_specs=[out_spec] * 2,
        out_shape=[jax.ShapeDtypeStruct((_VH, 2 * _D), jnp.float32)] * 2,
    )(t0t, t0t, t1t, t1t, t2t, t2t, itt, itt)


def _sc_gather_score(users, pos, neg, utab, itab):
    mesh = plsc.VectorSubcoreMesh(core_axis_name="c", subcore_axis_name="s")

    @functools.partial(
        pl.kernel,
        out_type=(
            jax.ShapeDtypeStruct((_B, _L), jnp.float32),
            jax.ShapeDtypeStruct((_NW, _L), jnp.float32),
        ),
        mesh=mesh,
        compiler_params=pltpu.CompilerParams(use_tc_tiling_on_sc=True,
                                             needs_layout_passes=False),
        scratch_types=[
            pltpu.VMEM((_BPW,), jnp.int32),
            pltpu.VMEM((_BPW,), jnp.int32),
            pltpu.VMEM((_BPW,), jnp.int32),
            pltpu.VMEM((_CHUNK, 2 * _D), jnp.float32),
            pltpu.VMEM((_CHUNK, 2 * _D), jnp.float32),
            pltpu.VMEM((_CHUNK, 2 * _D), jnp.float32),
            pltpu.VMEM((_BPW, _L), jnp.float32),
            pltpu.VMEM((_L,), jnp.float32),
            pltpu.SemaphoreType.DMA,
        ],
    )
    def sc_body(users_h, pos_h, neg_h, utab_h, itab_h,
                q_h, r_h,
                uidx, pidx, nidx, uv, pv, nv, qv, regv, sem):
        wid = lax.axis_index("s") * _NC + lax.axis_index("c")
        base = wid * _BPW
        pltpu.sync_copy(users_h.at[pl.ds(base, _BPW)], uidx)
        pltpu.sync_copy(pos_h.at[pl.ds(base, _BPW)], pidx)
        pltpu.sync_copy(neg_h.at[pl.ds(base, _BPW)], nidx)

        def chunk_body(ci, regacc):
            off = ci * _CHUNK
            for g in range(_CHUNK // _L):
                sl16 = pl.ds(off + g * _L, _L)
                uvec = uidx[sl16]
                pvec = pidx[sl16]
                nvec = nidx[sl16]
                uvec = uvec - jnp.where(uvec >= _VH, _VH, 0)
                pvec = pvec - jnp.where(pvec >= _VH, _VH, 0)
                nvec = nvec - jnp.where(nvec >= _VH, _VH, 0)
                for l in range(_L):
                    j = g * _L + l
                    pltpu.async_copy(utab_h.at[uvec[l]], uv.at[j], sem)
                    pltpu.async_copy(itab_h.at[pvec[l]], pv.at[j], sem)
                    pltpu.async_copy(itab_h.at[nvec[l]], nv.at[j], sem)
            pltpu.make_async_copy(utab_h.at[pl.ds(0, _CHUNK)], uv, sem).wait()
            pltpu.make_async_copy(itab_h.at[pl.ds(0, _CHUNK)], pv, sem).wait()
            pltpu.make_async_copy(itab_h.at[pl.ds(0, _CHUNK)], nv, sem).wait()

            lane = lax.iota(jnp.int32, _L)

            def row_body(r, reg):
                rsplat = jnp.full((_L,), r, jnp.int32)
                isplat = rsplat + off
                cu = jnp.where(plsc.load_gather(uidx, [isplat]) >= _VH,
                               _D, 0) + lane
                cp = jnp.where(plsc.load_gather(pidx, [isplat]) >= _VH,
                               _D, 0) + lane
                cn = jnp.where(plsc.load_gather(nidx, [isplat]) >= _VH,
                               _D, 0) + lane
                q = None
                for k in range(_NBLK):
                    u = plsc.load_gather(uv, [rsplat, cu + k * _L])
                    pp = plsc.load_gather(pv, [rsplat, cp + k * _L])
                    nn = plsc.load_gather(nv, [rsplat, cn + k * _L])
                    term = u * (nn - pp)
                    q = term if q is None else q + term
                    reg = reg + (u * u + pp * pp + nn * nn)
                qv[off + r, :] = q
                return reg

            return lax.fori_loop(0, _CHUNK, row_body, regacc)

        regacc = lax.fori_loop(0, _NCHUNK, chunk_body,
                               jnp.zeros((_L,), jnp.float32))
        pltpu.sync_copy(qv, q_h.at[pl.ds(base, _BPW)])
        regv[...] = regacc
        pltpu.sync_copy(regv, r_h.at[wid])

    return sc_body(users, pos, neg, utab, itab)


def _tc_finalize(q, r):
    def tc_body(q_ref, r_ref, loss_ref, reg_ref):
        d = jnp.sum(q_ref[...], axis=1)
        loss_ref[0, 0] = jnp.mean(jax.nn.softplus(d))
        reg_ref[0, 0] = 0.5 * jnp.sum(r_ref[...]) / float(_B)

    loss, reg = pl.pallas_call(
        tc_body,
        out_shape=(
            jax.ShapeDtypeStruct((1, 1), jnp.float32),
            jax.ShapeDtypeStruct((1, 1), jnp.float32),
        ),
        out_specs=(
            pl.BlockSpec(memory_space=pltpu.SMEM),
            pl.BlockSpec(memory_space=pltpu.SMEM),
        ),
    )(q, r)
    return loss[0, 0], reg[0, 0]


def kernel(users, pos, neg, user_table_0, user_table_1, user_table_2,
           item_table):
    utab, itab = _tc_pack(user_table_0, user_table_1, user_table_2,
                          item_table)
    q, r = _sc_gather_score(users, pos, neg, utab, itab)
    loss, reg_loss = _tc_finalize(q, r)
    return (loss, reg_loss)


# speedup vs baseline: 1.9601x; 1.1286x over previous
import functools

import jax
import jax.numpy as jnp
from jax import lax
from jax.experimental import pallas as pl
from jax.experimental.pallas import tpu as pltpu
from jax.experimental.pallas import tpu_sc as plsc

_B = 16384
_V = 100000
_D = 64
_L = 16
_NC = 2
_NS = 16
_NW = _NC * _NS
_BPW = _B // _NW
_CHUNK = 128
_NCHUNK = _BPW // _CHUNK
_NBLK = _D // _L
_CBH = 2048
_VH = 51200


def _tc_pack(t0, t1, t2, item):
    t0t, t1t, t2t, itt = t0.T, t1.T, t2.T, item.T

    def pack_body(a0l, a0h, a1l, a1h, a2l, a2h, itl, ith, u_ref, i_ref):
        ul = a0l[...] + 0.5 * (a1l[...] + a2l[...])
        uh = a0h[...] + 0.5 * (a1h[...] + a2h[...])
        u_ref[...] = jnp.concatenate([ul.T, uh.T], axis=1)
        i_ref[...] = jnp.concatenate([itl[...].T, ith[...].T], axis=1)

    nblk = _VH // _CBH
    last_in_blk = pl.cdiv(_V, _CBH) - 1
    lo_spec = pl.BlockSpec((_D, _CBH), lambda j: (0, j))
    hi_spec = pl.BlockSpec(
        (_D, _CBH), lambda j: (0, jnp.minimum(j + nblk, last_in_blk)))
    out_spec = pl.BlockSpec((_CBH, 2 * _D), lambda j: (j, 0))
    return pl.pallas_call(
        pack_body,
        grid=(nblk,),
        in_specs=[lo_spec, hi_spec] * 4,
        out_specs=[out_spec] * 2,
        out_shape=[jax.ShapeDtypeStruct((_VH, 2 * _D), jnp.float32)] * 2,
    )(t0t, t0t, t1t, t1t, t2t, t2t, itt, itt)


def _sc_gather_score(users, pos, neg, utab, itab):
    mesh = plsc.VectorSubcoreMesh(core_axis_name="c", subcore_axis_name="s")

    rowbuf = pltpu.VMEM((_CHUNK, 2 * _D), jnp.float32)

    @functools.partial(
        pl.kernel,
        out_type=(
            jax.ShapeDtypeStruct((_L, _B), jnp.float32),
            jax.ShapeDtypeStruct((_NW, _L), jnp.float32),
        ),
        mesh=mesh,
        compiler_params=pltpu.CompilerParams(use_tc_tiling_on_sc=True,
                                             needs_layout_passes=False),
        scratch_types=[
            pltpu.VMEM((_BPW,), jnp.int32),
            pltpu.VMEM((_BPW,), jnp.int32),
            pltpu.VMEM((_BPW,), jnp.int32),
            rowbuf, rowbuf, rowbuf,
            rowbuf, rowbuf, rowbuf,
            pltpu.VMEM((_L, _BPW), jnp.float32),
            pltpu.VMEM((_L,), jnp.float32),
            pltpu.SemaphoreType.DMA,
            pltpu.SemaphoreType.DMA,
        ],
    )
    def sc_body(users_h, pos_h, neg_h, utab_h, itab_h,
                q_h, r_h,
                uidx, pidx, nidx,
                uva, pva, nva, uvb, pvb, nvb,
                qv, regv, sema, semb):
        wid = lax.axis_index("s") * _NC + lax.axis_index("c")
        base = wid * _BPW
        pltpu.sync_copy(users_h.at[pl.ds(base, _BPW)], uidx)
        pltpu.sync_copy(pos_h.at[pl.ds(base, _BPW)], pidx)
        pltpu.sync_copy(neg_h.at[pl.ds(base, _BPW)], nidx)

        lane = lax.iota(jnp.int32, _L)

        def fire(ci, uv, pv, nv, sem):
            def g_body(g, carry):
                off = ci * _CHUNK + g * _L
                sl16 = pl.ds(off, _L)
                uvec = uidx[sl16]
                pvec = pidx[sl16]
                nvec = nidx[sl16]
                uvec = uvec - jnp.where(uvec >= _VH, _VH, 0)
                pvec = pvec - jnp.where(pvec >= _VH, _VH, 0)
                nvec = nvec - jnp.where(nvec >= _VH, _VH, 0)
                jg = g * _L
                for l in range(_L):
                    pltpu.async_copy(utab_h.at[uvec[l]], uv.at[jg + l], sem)
                    pltpu.async_copy(itab_h.at[pvec[l]], pv.at[jg + l], sem)
                    pltpu.async_copy(itab_h.at[nvec[l]], nv.at[jg + l], sem)
                return carry

            lax.fori_loop(0, _CHUNK // _L, g_body, 0)

        def drain(uv, pv, nv, sem):
            pltpu.make_async_copy(utab_h.at[pl.ds(0, _CHUNK)], uv, sem).wait()
            pltpu.make_async_copy(itab_h.at[pl.ds(0, _CHUNK)], pv, sem).wait()
            pltpu.make_async_copy(itab_h.at[pl.ds(0, _CHUNK)], nv, sem).wait()

        def compute(ci, uv, pv, nv, regacc):
            off = ci * _CHUNK

            def row_body(r, reg):
                rsplat = jnp.full((_L,), r, jnp.int32)
                isplat = rsplat + off
                cu = jnp.where(plsc.load_gather(uidx, [isplat]) >= _VH,
                               _D, 0) + lane
                cp = jnp.where(plsc.load_gather(pidx, [isplat]) >= _VH,
                               _D, 0) + lane
                cn = jnp.where(plsc.load_gather(nidx, [isplat]) >= _VH,
                               _D, 0) + lane
                q = None
                for k in range(_NBLK):
                    u = plsc.load_gather(uv, [rsplat, cu + k * _L])
                    pp = plsc.load_gather(pv, [rsplat, cp + k * _L])
                    nn = plsc.load_gather(nv, [rsplat, cn + k * _L])
                    term = u * (nn - pp)
                    q = term if q is None else q + term
                    reg = reg + (u * u + pp * pp + nn * nn)
                plsc.store_scatter(qv, [lane, isplat], q)
                return reg

            return lax.fori_loop(0, _CHUNK, row_body, regacc)

        regacc = jnp.zeros((_L,), jnp.float32)
        fire(0, uva, pva, nva, sema)
        fire(1, uvb, pvb, nvb, semb)
        drain(uva, pva, nva, sema)
        regacc = compute(0, uva, pva, nva, regacc)
        fire(2, uva, pva, nva, sema)
        drain(uvb, pvb, nvb, semb)
        regacc = compute(1, uvb, pvb, nvb, regacc)
        fire(3, uvb, pvb, nvb, semb)
        drain(uva, pva, nva, sema)
        regacc = compute(2, uva, pva, nva, regacc)
        drain(uvb, pvb, nvb, semb)
        regacc = compute(3, uvb, pvb, nvb, regacc)

        pltpu.sync_copy(qv, q_h.at[:, pl.ds(base, _BPW)])
        regv[...] = regacc
        pltpu.sync_copy(regv, r_h.at[wid])

    return sc_body(users, pos, neg, utab, itab)


def _tc_finalize(q, r):
    def tc_body(q_ref, r_ref, loss_ref, reg_ref):
        d = jnp.sum(q_ref[...], axis=0)
        loss_ref[0, 0] = jnp.mean(jax.nn.softplus(d))
        reg_ref[0, 0] = 0.5 * jnp.sum(r_ref[...]) / float(_B)

    loss, reg = pl.pallas_call(
        tc_body,
        out_shape=(
            jax.ShapeDtypeStruct((1, 1), jnp.float32),
            jax.ShapeDtypeStruct((1, 1), jnp.float32),
        ),
        out_specs=(
            pl.BlockSpec(memory_space=pltpu.SMEM),
            pl.BlockSpec(memory_space=pltpu.SMEM),
        ),
    )(q, r)
    return loss[0, 0], reg[0, 0]


def kernel(users, pos, neg, user_table_0, user_table_1, user_table_2,
           item_table):
    utab, itab = _tc_pack(user_table_0, user_table_1, user_table_2,
                          item_table)
    q, r = _sc_gather_score(users, pos, neg, utab, itab)
    loss, reg_loss = _tc_finalize(q, r)
    return (loss, reg_loss)


# speedup vs baseline: 2.0020x; 1.0214x over previous
import functools

import jax
import jax.numpy as jnp
from jax import lax
from jax.experimental import pallas as pl
from jax.experimental.pallas import tpu as pltpu
from jax.experimental.pallas import tpu_sc as plsc

_B = 16384
_V = 100000
_D = 64
_L = 16
_NC = 2
_NS = 16
_NW = _NC * _NS
_BPW = _B // _NW
_CHUNK = 128
_NCHUNK = _BPW // _CHUNK
_NBLK = _D // _L
_CBH = 2048
_VH = 51200


def _tc_pack(t0, t1, t2, item):
    t0t, t1t, t2t, itt = t0.T, t1.T, t2.T, item.T

    def pack_body(a0l, a0h, a1l, a1h, a2l, a2h, itl, ith, u_ref, i_ref):
        ul = a0l[...] + 0.5 * (a1l[...] + a2l[...])
        uh = a0h[...] + 0.5 * (a1h[...] + a2h[...])
        u_ref[...] = jnp.concatenate([ul.T, uh.T], axis=1)
        i_ref[...] = jnp.concatenate([itl[...].T, ith[...].T], axis=1)

    nblk = _VH // _CBH
    last_in_blk = pl.cdiv(_V, _CBH) - 1
    lo_spec = pl.BlockSpec((_D, _CBH), lambda j: (0, j))
    hi_spec = pl.BlockSpec(
        (_D, _CBH), lambda j: (0, jnp.minimum(j + nblk, last_in_blk)))
    out_spec = pl.BlockSpec((_CBH, 2 * _D), lambda j: (j, 0))
    return pl.pallas_call(
        pack_body,
        grid=(nblk,),
        in_specs=[lo_spec, hi_spec] * 4,
        out_specs=[out_spec] * 2,
        out_shape=[jax.ShapeDtypeStruct((_VH, 2 * _D), jnp.float32)] * 2,
    )(t0t, t0t, t1t, t1t, t2t, t2t, itt, itt)


def _sc_gather_score(users, pos, neg, utab, itab):
    mesh = plsc.VectorSubcoreMesh(core_axis_name="c", subcore_axis_name="s")

    rowbuf = pltpu.VMEM((_CHUNK, 2 * _D), jnp.float32)

    @functools.partial(
        pl.kernel,
        out_type=(
            jax.ShapeDtypeStruct((_L, _B), jnp.float32),
            jax.ShapeDtypeStruct((_NW, _L), jnp.float32),
        ),
        mesh=mesh,
        compiler_params=pltpu.CompilerParams(use_tc_tiling_on_sc=True,
                                             needs_layout_passes=False),
        scratch_types=[
            pltpu.VMEM((_BPW,), jnp.int32),
            pltpu.VMEM((_BPW,), jnp.int32),
            pltpu.VMEM((_BPW,), jnp.int32),
            pltpu.VMEM((_BPW,), jnp.int32),
            pltpu.VMEM((_BPW,), jnp.int32),
            pltpu.VMEM((_BPW,), jnp.int32),
            rowbuf, rowbuf, rowbuf,
            rowbuf, rowbuf, rowbuf,
            pltpu.VMEM((_L, _BPW), jnp.float32),
            pltpu.VMEM((_L,), jnp.float32),
            pltpu.SemaphoreType.DMA,
            pltpu.SemaphoreType.DMA,
        ],
    )
    def sc_body(users_h, pos_h, neg_h, utab_h, itab_h,
                q_h, r_h,
                uidx, pidx, nidx, muidx, mpidx, mnidx,
                uva, pva, nva, uvb, pvb, nvb,
                qv, regv, sema, semb):
        wid = lax.axis_index("s") * _NC + lax.axis_index("c")
        base = wid * _BPW
        pltpu.sync_copy(users_h.at[pl.ds(base, _BPW)], uidx)
        pltpu.sync_copy(pos_h.at[pl.ds(base, _BPW)], pidx)
        pltpu.sync_copy(neg_h.at[pl.ds(base, _BPW)], nidx)

        lane = lax.iota(jnp.int32, _L)

        def m_body(g, carry):
            sl16 = pl.ds(g * _L, _L)
            for src, dst in ((uidx, muidx), (pidx, mpidx), (nidx, mnidx)):
                v = src[sl16]
                dst[sl16] = v - jnp.where(v >= _VH, _VH, 0)
            return carry

        lax.fori_loop(0, _BPW // _L, m_body, 0)

        def fire(ci, uv, pv, nv, sem):
            isl = pl.ds(ci * _CHUNK, _CHUNK)
            pltpu.async_copy(utab_h.at[muidx.at[isl]], uv, sem)
            pltpu.async_copy(itab_h.at[mpidx.at[isl]], pv, sem)
            pltpu.async_copy(itab_h.at[mnidx.at[isl]], nv, sem)

        def drain(uv, pv, nv, sem):
            pltpu.make_async_copy(utab_h.at[pl.ds(0, _CHUNK)], uv, sem).wait()
            pltpu.make_async_copy(itab_h.at[pl.ds(0, _CHUNK)], pv, sem).wait()
            pltpu.make_async_copy(itab_h.at[pl.ds(0, _CHUNK)], nv, sem).wait()

        def compute(ci, uv, pv, nv, regacc):
            off = ci * _CHUNK

            def row_body(r, reg):
                rsplat = jnp.full((_L,), r, jnp.int32)
                isplat = rsplat + off
                cu = jnp.where(plsc.load_gather(uidx, [isplat]) >= _VH,
                               _D, 0) + lane
                cp = jnp.where(plsc.load_gather(pidx, [isplat]) >= _VH,
                               _D, 0) + lane
                cn = jnp.where(plsc.load_gather(nidx, [isplat]) >= _VH,
                               _D, 0) + lane
                q = None
                for k in range(_NBLK):
                    u = plsc.load_gather(uv, [rsplat, cu + k * _L])
                    pp = plsc.load_gather(pv, [rsplat, cp + k * _L])
                    nn = plsc.load_gather(nv, [rsplat, cn + k * _L])
                    term = u * (nn - pp)
                    q = term if q is None else q + term
                    reg = reg + (u * u + pp * pp + nn * nn)
                plsc.store_scatter(qv, [lane, isplat], q)
                return reg

            return lax.fori_loop(0, _CHUNK, row_body, regacc)

        regacc = jnp.zeros((_L,), jnp.float32)
        fire(0, uva, pva, nva, sema)
        fire(1, uvb, pvb, nvb, semb)
        drain(uva, pva, nva, sema)
        regacc = compute(0, uva, pva, nva, regacc)
        fire(2, uva, pva, nva, sema)
        drain(uvb, pvb, nvb, semb)
        regacc = compute(1, uvb, pvb, nvb, regacc)
        fire(3, uvb, pvb, nvb, semb)
        drain(uva, pva, nva, sema)
        regacc = compute(2, uva, pva, nva, regacc)
        drain(uvb, pvb, nvb, semb)
        regacc = compute(3, uvb, pvb, nvb, regacc)

        pltpu.sync_copy(qv, q_h.at[:, pl.ds(base, _BPW)])
        regv[...] = regacc
        pltpu.sync_copy(regv, r_h.at[wid])

    return sc_body(users, pos, neg, utab, itab)


def _tc_finalize(q, r):
    def tc_body(q_ref, r_ref, loss_ref, reg_ref):
        d = jnp.sum(q_ref[...], axis=0)
        loss_ref[0, 0] = jnp.mean(jax.nn.softplus(d))
        reg_ref[0, 0] = 0.5 * jnp.sum(r_ref[...]) / float(_B)

    loss, reg = pl.pallas_call(
        tc_body,
        out_shape=(
            jax.ShapeDtypeStruct((1, 1), jnp.float32),
            jax.ShapeDtypeStruct((1, 1), jnp.float32),
        ),
        out_specs=(
            pl.BlockSpec(memory_space=pltpu.SMEM),
            pl.BlockSpec(memory_space=pltpu.SMEM),
        ),
    )(q, r)
    return loss[0, 0], reg[0, 0]


def kernel(users, pos, neg, user_table_0, user_table_1, user_table_2,
           item_table):
    utab, itab = _tc_pack(user_table_0, user_table_1, user_table_2,
                          item_table)
    q, r = _sc_gather_score(users, pos, neg, utab, itab)
    loss, reg_loss = _tc_finalize(q, r)
    return (loss, reg_loss)


# speedup vs baseline: 2.1146x; 1.0562x over previous
import functools

import jax
import jax.numpy as jnp
from jax import lax
from jax.experimental import pallas as pl
from jax.experimental.pallas import tpu as pltpu
from jax.experimental.pallas import tpu_sc as plsc

_B = 16384
_V = 100000
_D = 64
_L = 16
_NC = 2
_NS = 16
_NW = _NC * _NS
_BPW = _B // _NW
_CHUNK = 128
_NCHUNK = _BPW // _CHUNK
_NBLK = _D // _L
_CBH = 4096
_VH = 53248


def _tc_pack(t0, t1, t2, item):
    t0t, t1t, t2t, itt = t0.T, t1.T, t2.T, item.T

    def pack_body(a0l, a0h, a1l, a1h, a2l, a2h, itl, ith, u_ref, i_ref):
        ul = a0l[...] + 0.5 * (a1l[...] + a2l[...])
        uh = a0h[...] + 0.5 * (a1h[...] + a2h[...])
        u_ref[...] = jnp.concatenate([ul.T, uh.T], axis=1)
        i_ref[...] = jnp.concatenate([itl[...].T, ith[...].T], axis=1)

    nblk = _VH // _CBH
    last_in_blk = pl.cdiv(_V, _CBH) - 1
    lo_spec = pl.BlockSpec((_D, _CBH), lambda j: (0, j))
    hi_spec = pl.BlockSpec(
        (_D, _CBH), lambda j: (0, jnp.minimum(j + nblk, last_in_blk)))
    out_spec = pl.BlockSpec((_CBH, 2 * _D), lambda j: (j, 0))
    return pl.pallas_call(
        pack_body,
        grid=(nblk,),
        in_specs=[lo_spec, hi_spec] * 4,
        out_specs=[out_spec] * 2,
        out_shape=[jax.ShapeDtypeStruct((_VH, 2 * _D), jnp.float32)] * 2,
    )(t0t, t0t, t1t, t1t, t2t, t2t, itt, itt)


def _sc_gather_score(users, pos, neg, utab, itab):
    mesh = plsc.VectorSubcoreMesh(core_axis_name="c", subcore_axis_name="s")

    rowbuf = pltpu.VMEM((_CHUNK, 2 * _D), jnp.float32)

    @functools.partial(
        pl.kernel,
        out_type=(
            jax.ShapeDtypeStruct((_L, _B), jnp.float32),
            jax.ShapeDtypeStruct((_NW, _L), jnp.float32),
        ),
        mesh=mesh,
        compiler_params=pltpu.CompilerParams(use_tc_tiling_on_sc=True,
                                             needs_layout_passes=False),
        scratch_types=[
            pltpu.VMEM((_BPW,), jnp.int32),
            pltpu.VMEM((_BPW,), jnp.int32),
            pltpu.VMEM((_BPW,), jnp.int32),
            pltpu.VMEM((_BPW,), jnp.int32),
            pltpu.VMEM((_BPW,), jnp.int32),
            pltpu.VMEM((_BPW,), jnp.int32),
            rowbuf, rowbuf, rowbuf,
            rowbuf, rowbuf, rowbuf,
            pltpu.VMEM((_L, _BPW), jnp.float32),
            pltpu.VMEM((_L,), jnp.float32),
            pltpu.SemaphoreType.DMA,
            pltpu.SemaphoreType.DMA,
        ],
    )
    def sc_body(users_h, pos_h, neg_h, utab_h, itab_h,
                q_h, r_h,
                uidx, pidx, nidx, muidx, mpidx, mnidx,
                uva, pva, nva, uvb, pvb, nvb,
                qv, regv, sema, semb):
        wid = lax.axis_index("s") * _NC + lax.axis_index("c")
        base = wid * _BPW
        pltpu.sync_copy(users_h.at[pl.ds(base, _BPW)], uidx)
        pltpu.sync_copy(pos_h.at[pl.ds(base, _BPW)], pidx)
        pltpu.sync_copy(neg_h.at[pl.ds(base, _BPW)], nidx)

        lane = lax.iota(jnp.int32, _L)

        def m_body(g, carry):
            sl16 = pl.ds(g * _L, _L)
            for src, dst in ((uidx, muidx), (pidx, mpidx), (nidx, mnidx)):
                v = src[sl16]
                dst[sl16] = v - jnp.where(v >= _VH, _VH, 0)
            return carry

        lax.fori_loop(0, _BPW // _L, m_body, 0)

        def fire(ci, uv, pv, nv, sem):
            isl = pl.ds(ci * _CHUNK, _CHUNK)
            pltpu.async_copy(utab_h.at[muidx.at[isl]], uv, sem)
            pltpu.async_copy(itab_h.at[mpidx.at[isl]], pv, sem)
            pltpu.async_copy(itab_h.at[mnidx.at[isl]], nv, sem)

        def drain(uv, pv, nv, sem):
            pltpu.make_async_copy(utab_h.at[pl.ds(0, _CHUNK)], uv, sem).wait()
            pltpu.make_async_copy(itab_h.at[pl.ds(0, _CHUNK)], pv, sem).wait()
            pltpu.make_async_copy(itab_h.at[pl.ds(0, _CHUNK)], nv, sem).wait()

        def compute(ci, uv, pv, nv, regacc):
            off = ci * _CHUNK

            def row_body(r, reg):
                rsplat = jnp.full((_L,), r, jnp.int32)
                isplat = rsplat + off
                cu = jnp.where(plsc.load_gather(uidx, [isplat]) >= _VH,
                               _D, 0) + lane
                cp = jnp.where(plsc.load_gather(pidx, [isplat]) >= _VH,
                               _D, 0) + lane
                cn = jnp.where(plsc.load_gather(nidx, [isplat]) >= _VH,
                               _D, 0) + lane
                q = None
                for k in range(_NBLK):
                    u = plsc.load_gather(uv, [rsplat, cu + k * _L])
                    pp = plsc.load_gather(pv, [rsplat, cp + k * _L])
                    nn = plsc.load_gather(nv, [rsplat, cn + k * _L])
                    term = u * (nn - pp)
                    q = term if q is None else q + term
                    reg = reg + (u * u + pp * pp + nn * nn)
                plsc.store_scatter(qv, [lane, isplat], q)
                return reg

            return lax.fori_loop(0, _CHUNK, row_body, regacc)

        regacc = jnp.zeros((_L,), jnp.float32)
        fire(0, uva, pva, nva, sema)
        fire(1, uvb, pvb, nvb, semb)
        drain(uva, pva, nva, sema)
        regacc = compute(0, uva, pva, nva, regacc)
        fire(2, uva, pva, nva, sema)
        drain(uvb, pvb, nvb, semb)
        regacc = compute(1, uvb, pvb, nvb, regacc)
        fire(3, uvb, pvb, nvb, semb)
        drain(uva, pva, nva, sema)
        regacc = compute(2, uva, pva, nva, regacc)
        drain(uvb, pvb, nvb, semb)
        regacc = compute(3, uvb, pvb, nvb, regacc)

        pltpu.sync_copy(qv, q_h.at[:, pl.ds(base, _BPW)])
        regv[...] = regacc
        pltpu.sync_copy(regv, r_h.at[wid])

    return sc_body(users, pos, neg, utab, itab)


def _tc_finalize(q, r):
    def tc_body(q_ref, r_ref, loss_ref, reg_ref):
        d = jnp.sum(q_ref[...], axis=0)
        loss_ref[0, 0] = jnp.mean(jax.nn.softplus(d))
        reg_ref[0, 0] = 0.5 * jnp.sum(r_ref[...]) / float(_B)

    loss, reg = pl.pallas_call(
        tc_body,
        out_shape=(
            jax.ShapeDtypeStruct((1, 1), jnp.float32),
            jax.ShapeDtypeStruct((1, 1), jnp.float32),
        ),
        out_specs=(
            pl.BlockSpec(memory_space=pltpu.SMEM),
            pl.BlockSpec(memory_space=pltpu.SMEM),
        ),
    )(q, r)
    return loss[0, 0], reg[0, 0]


def kernel(users, pos, neg, user_table_0, user_table_1, user_table_2,
           item_table):
    utab, itab = _tc_pack(user_table_0, user_table_1, user_table_2,
                          item_table)
    q, r = _sc_gather_score(users, pos, neg, utab, itab)
    loss, reg_loss = _tc_finalize(q, r)
    return (loss, reg_loss)


# speedup vs baseline: 2.1723x; 1.0273x over previous
import functools

import jax
import jax.numpy as jnp
from jax import lax
from jax.experimental import pallas as pl
from jax.experimental.pallas import tpu as pltpu
from jax.experimental.pallas import tpu_sc as plsc

_B = 16384
_V = 100000
_D = 64
_L = 16
_NC = 2
_NS = 16
_NW = _NC * _NS
_BPW = _B // _NW
_CHUNK = 128
_NCHUNK = _BPW // _CHUNK
_NBLK = _D // _L
_CBH = 4096
_VH = 53248


def _tc_pack(t0, t1, t2, item):
    t0t, t1t, t2t, itt = t0.T, t1.T, t2.T, item.T

    def pack_body(a0l, a0h, a1l, a1h, a2l, a2h, itl, ith, u_ref, i_ref):
        ul = a0l[...] + 0.5 * (a1l[...] + a2l[...])
        uh = a0h[...] + 0.5 * (a1h[...] + a2h[...])
        u_ref[...] = jnp.concatenate([ul.T, uh.T], axis=1)
        i_ref[...] = jnp.concatenate([itl[...].T, ith[...].T], axis=1)

    nblk = _VH // _CBH
    last_in_blk = pl.cdiv(_V, _CBH) - 1
    lo_spec = pl.BlockSpec((_D, _CBH), lambda j: (0, j))
    hi_spec = pl.BlockSpec(
        (_D, _CBH), lambda j: (0, jnp.minimum(j + nblk, last_in_blk)))
    out_spec = pl.BlockSpec((_CBH, 2 * _D), lambda j: (j, 0))
    return pl.pallas_call(
        pack_body,
        grid=(nblk,),
        in_specs=[lo_spec, hi_spec] * 4,
        out_specs=[out_spec] * 2,
        out_shape=[jax.ShapeDtypeStruct((_VH, 2 * _D), jnp.float32)] * 2,
    )(t0t, t0t, t1t, t1t, t2t, t2t, itt, itt)


def _sc_gather_score(users, pos, neg, utab, itab):
    mesh = plsc.VectorSubcoreMesh(core_axis_name="c", subcore_axis_name="s")

    rowbuf = pltpu.VMEM((_CHUNK, 2 * _D), jnp.float32)

    @functools.partial(
        pl.kernel,
        out_type=(
            jax.ShapeDtypeStruct((_L, _B), jnp.float32),
            jax.ShapeDtypeStruct((_NW, _L), jnp.float32),
        ),
        mesh=mesh,
        compiler_params=pltpu.CompilerParams(use_tc_tiling_on_sc=True,
                                             needs_layout_passes=False),
        scratch_types=[
            pltpu.VMEM((_BPW,), jnp.int32),
            pltpu.VMEM((_BPW,), jnp.int32),
            pltpu.VMEM((_BPW,), jnp.int32),
            pltpu.VMEM((_BPW,), jnp.int32),
            pltpu.VMEM((_BPW,), jnp.int32),
            pltpu.VMEM((_BPW,), jnp.int32),
            rowbuf, rowbuf, rowbuf,
            rowbuf, rowbuf, rowbuf,
            pltpu.VMEM((_L, _BPW), jnp.float32),
            pltpu.VMEM((_L,), jnp.float32),
            pltpu.SemaphoreType.DMA,
            pltpu.SemaphoreType.DMA,
        ],
    )
    def sc_body(users_h, pos_h, neg_h, utab_h, itab_h,
                q_h, r_h,
                uidx, pidx, nidx, muidx, mpidx, mnidx,
                uva, pva, nva, uvb, pvb, nvb,
                qv, regv, sema, semb):
        wid = lax.axis_index("s") * _NC + lax.axis_index("c")
        base = wid * _BPW
        pltpu.sync_copy(users_h.at[pl.ds(base, _BPW)], uidx)
        pltpu.sync_copy(pos_h.at[pl.ds(base, _BPW)], pidx)
        pltpu.sync_copy(neg_h.at[pl.ds(base, _BPW)], nidx)

        lane = lax.iota(jnp.int32, _L)

        def m_body(g, carry):
            sl16 = pl.ds(g * _L, _L)
            for src, dst in ((uidx, muidx), (pidx, mpidx), (nidx, mnidx)):
                v = src[sl16]
                dst[sl16] = v - jnp.where(v >= _VH, _VH, 0)
            return carry

        lax.fori_loop(0, _BPW // _L, m_body, 0)

        def fire(ci, uv, pv, nv, sem):
            isl = pl.ds(ci * _CHUNK, _CHUNK)
            pltpu.async_copy(utab_h.at[muidx.at[isl]], uv, sem)
            pltpu.async_copy(itab_h.at[mpidx.at[isl]], pv, sem)
            pltpu.async_copy(itab_h.at[mnidx.at[isl]], nv, sem)

        def drain(uv, pv, nv, sem):
            pltpu.make_async_copy(utab_h.at[pl.ds(0, _CHUNK)], uv, sem).wait()
            pltpu.make_async_copy(itab_h.at[pl.ds(0, _CHUNK)], pv, sem).wait()
            pltpu.make_async_copy(itab_h.at[pl.ds(0, _CHUNK)], nv, sem).wait()

        def compute(ci, uv, pv, nv, regacc):
            off = ci * _CHUNK

            def row_body(r, reg):
                rsplat = jnp.full((_L,), r, jnp.int32)
                isplat = rsplat + off
                cu = jnp.where(plsc.load_gather(uidx, [isplat]) >= _VH,
                               _D, 0) + lane
                cp = jnp.where(plsc.load_gather(pidx, [isplat]) >= _VH,
                               _D, 0) + lane
                cn = jnp.where(plsc.load_gather(nidx, [isplat]) >= _VH,
                               _D, 0) + lane
                q = None
                for k in range(_NBLK):
                    u = plsc.load_gather(uv, [rsplat, cu + k * _L])
                    pp = plsc.load_gather(pv, [rsplat, cp + k * _L])
                    nn = plsc.load_gather(nv, [rsplat, cn + k * _L])
                    term = u * (nn - pp)
                    q = term if q is None else q + term
                    reg = reg + (u * u + pp * pp + nn * nn)
                plsc.store_scatter(qv, [lane, isplat], q)
                return reg

            return plsc.parallel_loop(0, _CHUNK, unroll=4,
                                      carry=regacc)(row_body)

        regacc = jnp.zeros((_L,), jnp.float32)
        fire(0, uva, pva, nva, sema)
        fire(1, uvb, pvb, nvb, semb)
        drain(uva, pva, nva, sema)
        regacc = compute(0, uva, pva, nva, regacc)
        fire(2, uva, pva, nva, sema)
        drain(uvb, pvb, nvb, semb)
        regacc = compute(1, uvb, pvb, nvb, regacc)
        fire(3, uvb, pvb, nvb, semb)
        drain(uva, pva, nva, sema)
        regacc = compute(2, uva, pva, nva, regacc)
        drain(uvb, pvb, nvb, semb)
        regacc = compute(3, uvb, pvb, nvb, regacc)

        pltpu.sync_copy(qv, q_h.at[:, pl.ds(base, _BPW)])
        regv[...] = regacc
        pltpu.sync_copy(regv, r_h.at[wid])

    return sc_body(users, pos, neg, utab, itab)


def _tc_finalize(q, r):
    def tc_body(q_ref, r_ref, loss_ref, reg_ref):
        d = jnp.sum(q_ref[...], axis=0)
        loss_ref[0, 0] = jnp.mean(jax.nn.softplus(d))
        reg_ref[0, 0] = 0.5 * jnp.sum(r_ref[...]) / float(_B)

    loss, reg = pl.pallas_call(
        tc_body,
        out_shape=(
            jax.ShapeDtypeStruct((1, 1), jnp.float32),
            jax.ShapeDtypeStruct((1, 1), jnp.float32),
        ),
        out_specs=(
            pl.BlockSpec(memory_space=pltpu.SMEM),
            pl.BlockSpec(memory_space=pltpu.SMEM),
        ),
    )(q, r)
    return loss[0, 0], reg[0, 0]


def kernel(users, pos, neg, user_table_0, user_table_1, user_table_2,
           item_table):
    utab, itab = _tc_pack(user_table_0, user_table_1, user_table_2,
                          item_table)
    q, r = _sc_gather_score(users, pos, neg, utab, itab)
    loss, reg_loss = _tc_finalize(q, r)
    return (loss, reg_loss)
